# baseline (speedup 1.0000x reference)
import functools

import jax
import jax.numpy as jnp
from jax import lax
from jax.experimental import pallas as pl
from jax.experimental.pallas import tpu as pltpu
from jax.experimental.pallas import tpu_sc as plsc

N = 10000
E = 160000
NUM_TYPES = 100
NUM_GRAPHS = 16
R = 1000
NB = N // R

f32 = jnp.float32
i32 = jnp.int32


def _sc_mesh():
    return plsc.VectorSubcoreMesh(core_axis_name="c", subcore_axis_name="s")


EVEC = E // 16
VMAX = 313
EPW_PAD = VMAX * 16


@functools.cache
def _make_deg():
    @functools.partial(
        pl.kernel,
        out_type=jax.ShapeDtypeStruct((NB, 32, R), f32),
        mesh=_sc_mesh(),
        compiler_params=pltpu.CompilerParams(needs_layout_passes=False, use_tc_tiling_on_sc=False),
        scratch_types=[
            pltpu.VMEM((N,), f32),
            pltpu.VMEM((EPW_PAD,), i32),
        ],
    )
    def _deg_kernel(dst_hbm, out_hbm, acc_v, dst_v):
        c = lax.axis_index("c")
        s = lax.axis_index("s")
        wid = c * 16 + s

        def zbody(i, _):
            acc_v[pl.ds(i * 16, 16)] = jnp.zeros((16,), f32)
            return 0

        lax.fori_loop(0, N // 16, zbody, 0)

        b = 312 * wid + wid // 2
        n = 313 - (wid % 2)
        pltpu.sync_copy(dst_hbm.at[pl.ds(b * 16, EPW_PAD)], dst_v)

        ones = jnp.ones((16,), f32)

        def ebody(i, _):
            idx = dst_v[pl.ds(i * 16, 16)]
            plsc.addupdate_scatter(acc_v, [idx], ones)
            return 0

        lax.fori_loop(0, n, ebody, 0)
        for b in range(NB):
            pltpu.sync_copy(acc_v.at[pl.ds(b * R, R)], out_hbm.at[b, wid])

    return _deg_kernel


@functools.cache
def _make_agg(n_chunks, D, CH, NBUF):
    per_sc = n_chunks // 2
    EPT = E // 16
    ITER = EPT // CH
    RPT = N // 16
    P = 3
    assert ITER % NBUF == 0 and CH % 8 == 0 and CH <= 128 and P < NBUF

    @functools.partial(
        pl.kernel,
        out_type=jax.ShapeDtypeStruct((n_chunks * N, D), f32),
        mesh=_sc_mesh(),
        compiler_params=pltpu.CompilerParams(needs_layout_passes=False, use_tc_tiling_on_sc=False),
        scratch_types=[
            pltpu.VMEM((EPT,), i32),
            pltpu.VMEM((ITER, CH), i32),
            pltpu.VMEM((NBUF, CH, D), f32),
            pltpu.VMEM_SHARED((N, D), f32),
            pltpu.SemaphoreType.DMA((NBUF,)),
            pltpu.SemaphoreType.DMA((NBUF,)),
        ],
    )
    def _agg(src_hbm, dst_hbm, g_hbm, zero_hbm, out_hbm,
             src_v, dst_v, rows_v, acc_sh, sem_g, sem_s):
        c = lax.axis_index("c")
        s = lax.axis_index("s")

        def gather(i, b):
            pltpu.async_copy(g_hbm.at[src_v.at[pl.ds(i * CH, CH)]],
                             rows_v.at[b], sem_g.at[b])

        def wait_gather(b):
            pltpu.make_async_copy(g_hbm.at[pl.ds(0, CH)], rows_v.at[b],
                                  sem_g.at[b]).wait()

        def scatter(i, b):
            pltpu.async_copy(rows_v.at[b], acc_sh.at[dst_v.at[i]],
                             sem_s.at[b], add=True)

        def wait_scatter(b):
            pltpu.make_async_copy(rows_v.at[b], acc_sh.at[pl.ds(0, CH)],
                                  sem_s.at[b]).wait()

        for k in range(per_sc):
            chunk = c * per_sc + k
            pltpu.sync_copy(src_hbm.at[pl.ds(chunk * E + s * EPT, EPT)], src_v)
            pltpu.sync_copy(dst_hbm.at[s], dst_v)
            for b in range(P):
                gather(b, b)
            pltpu.sync_copy(zero_hbm, acc_sh.at[pl.ds(s * RPT, RPT)])
            plsc.subcore_barrier()

            def obody(io, _):
                for b in range(NBUF):
                    i = io * NBUF + b
                    bg = (b + P) % NBUF
                    wait_gather(b)
                    scatter(i, b)

                    @pl.when(i >= NBUF - P)
                    def _():
                        wait_scatter(bg)

                    @pl.when(i + P < ITER)
                    def _():
                        gather(i + P, bg)
                return 0

            lax.fori_loop(0, ITER // NBUF, obody, 0)
            for i in range(ITER - (NBUF - P), ITER):
                wait_scatter(i % NBUF)
            plsc.subcore_barrier()
            pltpu.sync_copy(acc_sh.at[pl.ds(s * RPT, RPT)],
                            out_hbm.at[pl.ds(chunk * N + s * RPT, RPT)])

    return _agg


def _prep_body(x_ref, degp_ref, emb_ref, g1_ref, dis_ref):
    xb = x_ref[...]
    ones32 = jnp.ones((32, 1), f32)
    deg = lax.dot_general(degp_ref[0], ones32,
                          (((0,), (0,)), ((), ())),
                          preferred_element_type=f32) + 1.0
    dis = lax.rsqrt(deg)
    ids = xb[:, 0:1].astype(i32)
    oh = (ids == jnp.arange(NUM_TYPES, dtype=i32)[None, :]).astype(f32)
    te = jnp.dot(oh, emb_ref[...], preferred_element_type=f32)
    half0 = jnp.concatenate([te, xb[:, 1:49]], axis=1)
    half1 = jnp.concatenate([xb[:, 49:128], jnp.zeros((R, 1), f32)], axis=1)
    g1_ref[0] = half0 * dis
    g1_ref[1] = half1 * dis
    dis_ref[...] = dis


_prep_call = pl.pallas_call(
    _prep_body,
    grid=(NB,),
    in_specs=[
        pl.BlockSpec((R, 128), lambda i: (i, 0)),
        pl.BlockSpec((1, 32, R), lambda i: (i, 0, 0)),
        pl.BlockSpec((NUM_TYPES, 32), lambda i: (0, 0)),
    ],
    out_specs=[
        pl.BlockSpec((2, R, 80), lambda i: (0, i, 0)),
        pl.BlockSpec((R, 1), lambda i: (i, 0)),
    ],
    out_shape=[
        jax.ShapeDtypeStruct((2, N, 80), f32),
        jax.ShapeDtypeStruct((N, 1), f32),
    ],
)


def _mm_body(a1_ref, g1_ref, dis_ref, W1_ref, b1_ref, W2_ref, out_ref):
    d = dis_ref[...]
    x0 = ((a1_ref[0] + g1_ref[0]) * d).astype(jnp.bfloat16)
    x1 = ((a1_ref[1] + g1_ref[1]) * d).astype(jnp.bfloat16)
    h1 = jnp.dot(x0, W1_ref[0:80, :], preferred_element_type=f32)
    h1 = h1 + jnp.dot(x1, W1_ref[80:160, :], preferred_element_type=f32)
    h1 = jnp.maximum(h1 + b1_ref[...], 0.0).astype(jnp.bfloat16)
    t = jnp.dot(h1, W2_ref[...], preferred_element_type=f32) * d
    for j in range(8):
        out_ref[j] = t[:, j * 128:(j + 1) * 128]


_mm_call = pl.pallas_call(
    _mm_body,
    grid=(NB,),
    in_specs=[
        pl.BlockSpec((2, R, 80), lambda i: (0, i, 0)),
        pl.BlockSpec((2, R, 80), lambda i: (0, i, 0)),
        pl.BlockSpec((R, 1), lambda i: (i, 0)),
        pl.BlockSpec((160, 2048), lambda i: (0, 0)),
        pl.BlockSpec((1, 2048), lambda i: (0, 0)),
        pl.BlockSpec((2048, 1024), lambda i: (0, 0)),
    ],
    out_specs=pl.BlockSpec((8, R, 128), lambda i: (0, i, 0)),
    out_shape=jax.ShapeDtypeStruct((8, N, 128), f32),
)


def _head_body(a2_ref, g2_ref, dis_ref, b2_ref, batch_ref,
               Wh1_ref, bh1_ref, Wh2_ref, bh2_ref, Wo_ref, bo_ref,
               out_ref, pool_s, cnt_s):
    i = pl.program_id(0)

    @pl.when(i == 0)
    def _():
        pool_s[...] = jnp.zeros_like(pool_s)
        cnt_s[...] = jnp.zeros_like(cnt_s)

    d = dis_ref[...]
    bvals = batch_ref[0]
    oh = (bvals == jnp.arange(NUM_GRAPHS, dtype=i32)[:, None]).astype(f32)
    cnt_s[...] += jnp.broadcast_to(
        jnp.sum(oh, axis=1, keepdims=True), cnt_s.shape)
    for j in range(8):
        h2 = (a2_ref[j] + g2_ref[j]) * d + b2_ref[0, j * 128:(j + 1) * 128]
        h2 = jnp.maximum(h2, 0.0)
        pool_s[:, j * 128:(j + 1) * 128] += jnp.dot(
            oh, h2, preferred_element_type=f32)

    @pl.when(i == NB - 1)
    def _():
        cnt = jnp.maximum(cnt_s[:, 0:1], 1.0)
        p = pool_s[...] / cnt
        hh = jnp.maximum(
            jnp.dot(p, Wh1_ref[...], preferred_element_type=f32)
            + bh1_ref[...], 0.0)
        hh = jnp.maximum(
            jnp.dot(hh, Wh2_ref[...], preferred_element_type=f32)
            + bh2_ref[...], 0.0)
        lo = jnp.dot(hh, Wo_ref[...], preferred_element_type=f32) + bo_ref[...]
        m = jnp.max(lo, axis=1, keepdims=True)
        e = jnp.exp(lo - m)
        out_ref[...] = e / jnp.sum(e, axis=1, keepdims=True)


_head_call = pl.pallas_call(
    _head_body,
    grid=(NB,),
    in_specs=[
        pl.BlockSpec((8, R, 128), lambda i: (0, i, 0)),
        pl.BlockSpec((8, R, 128), lambda i: (0, i, 0)),
        pl.BlockSpec((R, 1), lambda i: (i, 0)),
        pl.BlockSpec((1, 1024), lambda i: (0, 0)),
        pl.BlockSpec((1, 1, R), lambda i: (i, 0, 0)),
        pl.BlockSpec((1024, 1024), lambda i: (0, 0)),
        pl.BlockSpec((1, 1024), lambda i: (0, 0)),
        pl.BlockSpec((1024, 512), lambda i: (0, 0)),
        pl.BlockSpec((1, 512), lambda i: (0, 0)),
        pl.BlockSpec((512, 10), lambda i: (0, 0)),
        pl.BlockSpec((1, 10), lambda i: (0, 0)),
    ],
    out_specs=pl.BlockSpec((NUM_GRAPHS, 10), lambda i: (0, 0)),
    out_shape=jax.ShapeDtypeStruct((NUM_GRAPHS, 10), f32),
    scratch_shapes=[
        pltpu.VMEM((NUM_GRAPHS, 1024), f32),
        pltpu.VMEM((NUM_GRAPHS, 128), f32),
    ],
)


def kernel(x, edge_index, batch, emb_table, W1, b1, W2, b2,
           Wh1, bh1, Wh2, bh2, Wo, bo):
    src = edge_index[0]
    dst = edge_index[1]

    degp = _make_deg()(dst)
    g1, dis2 = _prep_call(x, degp, emb_table)

    off1 = (jnp.arange(2, dtype=i32) * N)[:, None]
    src1 = jnp.reshape(src[None, :] + off1, (-1,))
    a1 = _make_agg(2, 80, 80, 5)(src1, dst.reshape(16, 125, 80),
                                 g1.reshape(2 * N, 80),
                                 jnp.zeros((625, 80), f32))
    a1 = a1.reshape(2, N, 80)

    W1p = jnp.concatenate([W1, jnp.zeros((1, 2048), f32)], axis=0)
    g2 = _mm_call(a1, g1, dis2, W1p.astype(jnp.bfloat16),
                  b1.reshape(1, 2048), W2.astype(jnp.bfloat16))

    off2 = (jnp.arange(8, dtype=i32) * N)[:, None]
    src2 = jnp.reshape(src[None, :] + off2, (-1,))
    a2 = _make_agg(8, 128, 40, 5)(src2, dst.reshape(16, 250, 40),
                                  g2.reshape(8 * N, 128),
                                  jnp.zeros((625, 128), f32))
    a2 = a2.reshape(8, N, 128)

    return _head_call(a2, g2, dis2, b2.reshape(1, 1024),
                      batch.reshape(NB, 1, R), Wh1, bh1.reshape(1, 1024),
                      Wh2, bh2.reshape(1, 512), Wo, bo.reshape(1, 10))


# speedup vs baseline: 1.0720x; 1.0720x over previous
import functools

import jax
import jax.numpy as jnp
from jax import lax
from jax.experimental import pallas as pl
from jax.experimental.pallas import tpu as pltpu
from jax.experimental.pallas import tpu_sc as plsc

N = 10000
E = 160000
NUM_TYPES = 100
NUM_GRAPHS = 16
R = 1000
NB = N // R

f32 = jnp.float32
i32 = jnp.int32


def _sc_mesh():
    return plsc.VectorSubcoreMesh(core_axis_name="c", subcore_axis_name="s")


EVEC = E // 16
VMAX = 313
EPW_PAD = VMAX * 16


@functools.cache
def _make_deg():
    @functools.partial(
        pl.kernel,
        out_type=jax.ShapeDtypeStruct((NB, 32, R), f32),
        mesh=_sc_mesh(),
        compiler_params=pltpu.CompilerParams(needs_layout_passes=False, use_tc_tiling_on_sc=False),
        scratch_types=[
            pltpu.VMEM((N,), f32),
            pltpu.VMEM((EPW_PAD,), i32),
        ],
    )
    def _deg_kernel(dst_hbm, out_hbm, acc_v, dst_v):
        c = lax.axis_index("c")
        s = lax.axis_index("s")
        wid = c * 16 + s

        def zbody(i, _):
            acc_v[pl.ds(i * 16, 16)] = jnp.zeros((16,), f32)
            return 0

        lax.fori_loop(0, N // 16, zbody, 0)

        b = 312 * wid + wid // 2
        n = 313 - (wid % 2)
        pltpu.sync_copy(dst_hbm.at[pl.ds(b * 16, EPW_PAD)], dst_v)

        ones = jnp.ones((16,), f32)

        def ebody(i, _):
            idx = dst_v[pl.ds(i * 16, 16)]
            plsc.addupdate_scatter(acc_v, [idx], ones)
            return 0

        lax.fori_loop(0, n, ebody, 0)
        for b in range(NB):
            pltpu.sync_copy(acc_v.at[pl.ds(b * R, R)], out_hbm.at[b, wid])

    return _deg_kernel


@functools.cache
def _make_agg(n_chunks, D, CH, NBUF, dtype=f32):
    per_sc = n_chunks // 2
    EPT = E // 16
    ITER = EPT // CH
    RPT = N // 16
    P = 3
    assert ITER % NBUF == 0 and CH % 8 == 0 and CH <= 128 and P < NBUF

    @functools.partial(
        pl.kernel,
        out_type=jax.ShapeDtypeStruct((n_chunks * N, D), dtype),
        mesh=_sc_mesh(),
        compiler_params=pltpu.CompilerParams(needs_layout_passes=False, use_tc_tiling_on_sc=False),
        scratch_types=[
            pltpu.VMEM((EPT,), i32),
            pltpu.VMEM((ITER, CH), i32),
            pltpu.VMEM((NBUF, CH, D), dtype),
            pltpu.VMEM_SHARED((N, D), dtype),
            pltpu.SemaphoreType.DMA((NBUF,)),
            pltpu.SemaphoreType.DMA((NBUF,)),
        ],
    )
    def _agg(src_hbm, dst_hbm, g_hbm, zero_hbm, out_hbm,
             src_v, dst_v, rows_v, acc_sh, sem_g, sem_s):
        c = lax.axis_index("c")
        s = lax.axis_index("s")

        def gather(i, b):
            pltpu.async_copy(g_hbm.at[src_v.at[pl.ds(i * CH, CH)]],
                             rows_v.at[b], sem_g.at[b])

        def wait_gather(b):
            pltpu.make_async_copy(g_hbm.at[pl.ds(0, CH)], rows_v.at[b],
                                  sem_g.at[b]).wait()

        def scatter(i, b):
            pltpu.async_copy(rows_v.at[b], acc_sh.at[dst_v.at[i]],
                             sem_s.at[b], add=True)

        def wait_scatter(b):
            pltpu.make_async_copy(rows_v.at[b], acc_sh.at[pl.ds(0, CH)],
                                  sem_s.at[b]).wait()

        for k in range(per_sc):
            chunk = c * per_sc + k
            pltpu.sync_copy(src_hbm.at[pl.ds(chunk * E + s * EPT, EPT)], src_v)
            pltpu.sync_copy(dst_hbm.at[s], dst_v)
            for b in range(P):
                gather(b, b)
            pltpu.sync_copy(zero_hbm, acc_sh.at[pl.ds(s * RPT, RPT)])
            plsc.subcore_barrier()

            def obody(io, _):
                for b in range(NBUF):
                    i = io * NBUF + b
                    bg = (b + P) % NBUF
                    wait_gather(b)
                    scatter(i, b)

                    @pl.when(i >= NBUF - P)
                    def _():
                        wait_scatter(bg)

                    @pl.when(i + P < ITER)
                    def _():
                        gather(i + P, bg)
                return 0

            lax.fori_loop(0, ITER // NBUF, obody, 0)
            for i in range(ITER - (NBUF - P), ITER):
                wait_scatter(i % NBUF)
            plsc.subcore_barrier()
            pltpu.sync_copy(acc_sh.at[pl.ds(s * RPT, RPT)],
                            out_hbm.at[pl.ds(chunk * N + s * RPT, RPT)])

    return _agg


def _prep_body(x_ref, degp_ref, emb_ref, g1_ref, dis_ref):
    xb = x_ref[...]
    ones32 = jnp.ones((32, 1), f32)
    deg = lax.dot_general(degp_ref[0], ones32,
                          (((0,), (0,)), ((), ())),
                          preferred_element_type=f32) + 1.0
    dis = lax.rsqrt(deg)
    ids = xb[:, 0:1].astype(i32)
    oh = (ids == jnp.arange(NUM_TYPES, dtype=i32)[None, :]).astype(f32)
    te = jnp.dot(oh, emb_ref[...], preferred_element_type=f32)
    half0 = jnp.concatenate([te, xb[:, 1:49]], axis=1)
    half1 = jnp.concatenate([xb[:, 49:128], jnp.zeros((R, 1), f32)], axis=1)
    g1_ref[0] = half0 * dis
    g1_ref[1] = half1 * dis
    dis_ref[...] = dis


_prep_call = pl.pallas_call(
    _prep_body,
    grid=(NB,),
    in_specs=[
        pl.BlockSpec((R, 128), lambda i: (i, 0)),
        pl.BlockSpec((1, 32, R), lambda i: (i, 0, 0)),
        pl.BlockSpec((NUM_TYPES, 32), lambda i: (0, 0)),
    ],
    out_specs=[
        pl.BlockSpec((2, R, 80), lambda i: (0, i, 0)),
        pl.BlockSpec((R, 1), lambda i: (i, 0)),
    ],
    out_shape=[
        jax.ShapeDtypeStruct((2, N, 80), f32),
        jax.ShapeDtypeStruct((N, 1), f32),
    ],
)


def _mm_body(a1_ref, g1_ref, dis_ref, W1_ref, b1_ref, W2_ref, out_ref):
    d = dis_ref[...]
    x0 = ((a1_ref[0] + g1_ref[0]) * d).astype(jnp.bfloat16)
    x1 = ((a1_ref[1] + g1_ref[1]) * d).astype(jnp.bfloat16)
    h1 = jnp.dot(x0, W1_ref[0:80, :], preferred_element_type=f32)
    h1 = h1 + jnp.dot(x1, W1_ref[80:160, :], preferred_element_type=f32)
    h1 = jnp.maximum(h1 + b1_ref[...], 0.0).astype(jnp.bfloat16)
    t = (jnp.dot(h1, W2_ref[...], preferred_element_type=f32) * d).astype(
        jnp.bfloat16)
    for j in range(8):
        out_ref[j] = t[:, j * 128:(j + 1) * 128]


_mm_call = pl.pallas_call(
    _mm_body,
    grid=(NB,),
    in_specs=[
        pl.BlockSpec((2, R, 80), lambda i: (0, i, 0)),
        pl.BlockSpec((2, R, 80), lambda i: (0, i, 0)),
        pl.BlockSpec((R, 1), lambda i: (i, 0)),
        pl.BlockSpec((160, 2048), lambda i: (0, 0)),
        pl.BlockSpec((1, 2048), lambda i: (0, 0)),
        pl.BlockSpec((2048, 1024), lambda i: (0, 0)),
    ],
    out_specs=pl.BlockSpec((8, R, 128), lambda i: (0, i, 0)),
    out_shape=jax.ShapeDtypeStruct((8, N, 128), jnp.bfloat16),
)


def _head_body(a2_ref, g2_ref, dis_ref, b2_ref, batch_ref,
               Wh1_ref, bh1_ref, Wh2_ref, bh2_ref, Wo_ref, bo_ref,
               out_ref, pool_s, cnt_s):
    i = pl.program_id(0)

    @pl.when(i == 0)
    def _():
        pool_s[...] = jnp.zeros_like(pool_s)
        cnt_s[...] = jnp.zeros_like(cnt_s)

    d = dis_ref[...]
    bvals = batch_ref[0]
    oh = (bvals == jnp.arange(NUM_GRAPHS, dtype=i32)[:, None]).astype(f32)
    cnt_s[...] += jnp.broadcast_to(
        jnp.sum(oh, axis=1, keepdims=True), cnt_s.shape)
    for j in range(8):
        h2 = (a2_ref[j].astype(f32) + g2_ref[j].astype(f32)) * d
        h2 = jnp.maximum(h2 + b2_ref[0, j * 128:(j + 1) * 128], 0.0)
        pool_s[:, j * 128:(j + 1) * 128] += jnp.dot(
            oh, h2, preferred_element_type=f32)

    @pl.when(i == NB - 1)
    def _():
        cnt = jnp.maximum(cnt_s[:, 0:1], 1.0)
        p = pool_s[...] / cnt
        hh = jnp.maximum(
            jnp.dot(p, Wh1_ref[...], preferred_element_type=f32)
            + bh1_ref[...], 0.0)
        hh = jnp.maximum(
            jnp.dot(hh, Wh2_ref[...], preferred_element_type=f32)
            + bh2_ref[...], 0.0)
        lo = jnp.dot(hh, Wo_ref[...], preferred_element_type=f32) + bo_ref[...]
        m = jnp.max(lo, axis=1, keepdims=True)
        e = jnp.exp(lo - m)
        out_ref[...] = e / jnp.sum(e, axis=1, keepdims=True)


_head_call = pl.pallas_call(
    _head_body,
    grid=(NB,),
    in_specs=[
        pl.BlockSpec((8, R, 128), lambda i: (0, i, 0)),
        pl.BlockSpec((8, R, 128), lambda i: (0, i, 0)),
        pl.BlockSpec((R, 1), lambda i: (i, 0)),
        pl.BlockSpec((1, 1024), lambda i: (0, 0)),
        pl.BlockSpec((1, 1, R), lambda i: (i, 0, 0)),
        pl.BlockSpec((1024, 1024), lambda i: (0, 0)),
        pl.BlockSpec((1, 1024), lambda i: (0, 0)),
        pl.BlockSpec((1024, 512), lambda i: (0, 0)),
        pl.BlockSpec((1, 512), lambda i: (0, 0)),
        pl.BlockSpec((512, 10), lambda i: (0, 0)),
        pl.BlockSpec((1, 10), lambda i: (0, 0)),
    ],
    out_specs=pl.BlockSpec((NUM_GRAPHS, 10), lambda i: (0, 0)),
    out_shape=jax.ShapeDtypeStruct((NUM_GRAPHS, 10), f32),
    scratch_shapes=[
        pltpu.VMEM((NUM_GRAPHS, 1024), f32),
        pltpu.VMEM((NUM_GRAPHS, 128), f32),
    ],
)


def kernel(x, edge_index, batch, emb_table, W1, b1, W2, b2,
           Wh1, bh1, Wh2, bh2, Wo, bo):
    src = edge_index[0]
    dst = edge_index[1]

    degp = _make_deg()(dst)
    g1, dis2 = _prep_call(x, degp, emb_table)

    off1 = (jnp.arange(2, dtype=i32) * N)[:, None]
    src1 = jnp.reshape(src[None, :] + off1, (-1,))
    a1 = _make_agg(2, 80, 80, 5)(src1, dst.reshape(16, 125, 80),
                                 g1.reshape(2 * N, 80),
                                 jnp.zeros((625, 80), f32))
    a1 = a1.reshape(2, N, 80)

    W1p = jnp.concatenate([W1, jnp.zeros((1, 2048), f32)], axis=0)
    g2 = _mm_call(a1, g1, dis2, W1p.astype(jnp.bfloat16),
                  b1.reshape(1, 2048), W2.astype(jnp.bfloat16))

    off2 = (jnp.arange(8, dtype=i32) * N)[:, None]
    src2 = jnp.reshape(src[None, :] + off2, (-1,))
    a2 = _make_agg(8, 128, 80, 5, jnp.bfloat16)(
        src2, dst.reshape(16, 125, 80), g2.reshape(8 * N, 128),
        jnp.zeros((625, 128), jnp.bfloat16))
    a2 = a2.reshape(8, N, 128)

    return _head_call(a2, g2, dis2, b2.reshape(1, 1024),
                      batch.reshape(NB, 1, R), Wh1, bh1.reshape(1, 1024),
                      Wh2, bh2.reshape(1, 512), Wo, bo.reshape(1, 10))


# speedup vs baseline: 1.0970x; 1.0234x over previous
import functools

import jax
import jax.numpy as jnp
from jax import lax
from jax.experimental import pallas as pl
from jax.experimental.pallas import tpu as pltpu
from jax.experimental.pallas import tpu_sc as plsc

N = 10000
E = 160000
NUM_TYPES = 100
NUM_GRAPHS = 16
R = 1000
NB = N // R

f32 = jnp.float32
i32 = jnp.int32


def _sc_mesh():
    return plsc.VectorSubcoreMesh(core_axis_name="c", subcore_axis_name="s")


EVEC = E // 16
VMAX = 313
EPW_PAD = VMAX * 16


@functools.cache
def _make_deg():
    @functools.partial(
        pl.kernel,
        out_type=jax.ShapeDtypeStruct((NB, 32, R), f32),
        mesh=_sc_mesh(),
        compiler_params=pltpu.CompilerParams(needs_layout_passes=False, use_tc_tiling_on_sc=False),
        scratch_types=[
            pltpu.VMEM((N,), f32),
            pltpu.VMEM((EPW_PAD,), i32),
        ],
    )
    def _deg_kernel(dst_hbm, out_hbm, acc_v, dst_v):
        c = lax.axis_index("c")
        s = lax.axis_index("s")
        wid = c * 16 + s

        def zbody(i, _):
            acc_v[pl.ds(i * 16, 16)] = jnp.zeros((16,), f32)
            return 0

        lax.fori_loop(0, N // 16, zbody, 0)

        b = 312 * wid + wid // 2
        n = 313 - (wid % 2)
        pltpu.sync_copy(dst_hbm.at[pl.ds(b * 16, EPW_PAD)], dst_v)

        ones = jnp.ones((16,), f32)

        def ebody(i, _):
            idx = dst_v[pl.ds(i * 16, 16)]
            plsc.addupdate_scatter(acc_v, [idx], ones)
            return 0

        lax.fori_loop(0, n, ebody, 0)
        for b in range(NB):
            pltpu.sync_copy(acc_v.at[pl.ds(b * R, R)], out_hbm.at[b, wid])

    return _deg_kernel


@functools.cache
def _make_agg(n_chunks, D, CH, NBUF, dtype=f32):
    per_sc = n_chunks // 2
    EPT = E // 16
    ITER = EPT // CH
    RPT = N // 16
    P = 3
    assert ITER % NBUF == 0 and CH % 8 == 0 and CH <= 128 and P < NBUF

    @functools.partial(
        pl.kernel,
        out_type=jax.ShapeDtypeStruct((n_chunks * N, D), dtype),
        mesh=_sc_mesh(),
        compiler_params=pltpu.CompilerParams(needs_layout_passes=False, use_tc_tiling_on_sc=False),
        scratch_types=[
            pltpu.VMEM((EPT,), i32),
            pltpu.VMEM((ITER, CH), i32),
            pltpu.VMEM((NBUF, CH, D), dtype),
            pltpu.VMEM_SHARED((N, D), dtype),
            pltpu.SemaphoreType.DMA((NBUF,)),
            pltpu.SemaphoreType.DMA((NBUF,)),
        ],
    )
    def _agg(src_hbm, dst_hbm, g_hbm, out_hbm,
             src_v, dst_v, rows_v, acc_sh, sem_g, sem_s):
        c = lax.axis_index("c")
        s = lax.axis_index("s")

        def gather(i, b):
            pltpu.async_copy(g_hbm.at[src_v.at[pl.ds(i * CH, CH)]],
                             rows_v.at[b], sem_g.at[b])

        def wait_gather(b):
            pltpu.make_async_copy(g_hbm.at[pl.ds(0, CH)], rows_v.at[b],
                                  sem_g.at[b]).wait()

        def scatter(i, b):
            pltpu.async_copy(rows_v.at[b], acc_sh.at[dst_v.at[i]],
                             sem_s.at[b], add=True)

        def wait_scatter(b):
            pltpu.make_async_copy(rows_v.at[b], acc_sh.at[pl.ds(0, CH)],
                                  sem_s.at[b]).wait()

        for k in range(per_sc):
            chunk = c * per_sc + k
            pltpu.sync_copy(src_hbm.at[pl.ds(chunk * E + s * EPT, EPT)], src_v)
            pltpu.sync_copy(dst_hbm.at[s], dst_v)
            for b in range(P):
                gather(b, b)
            pltpu.sync_copy(g_hbm.at[pl.ds(chunk * N + s * RPT, RPT)],
                            acc_sh.at[pl.ds(s * RPT, RPT)])
            plsc.subcore_barrier()

            def obody(io, _):
                for b in range(NBUF):
                    i = io * NBUF + b
                    bg = (b + P) % NBUF
                    wait_gather(b)
                    scatter(i, b)

                    @pl.when(i >= NBUF - P)
                    def _():
                        wait_scatter(bg)

                    @pl.when(i + P < ITER)
                    def _():
                        gather(i + P, bg)
                return 0

            lax.fori_loop(0, ITER // NBUF, obody, 0)
            for i in range(ITER - (NBUF - P), ITER):
                wait_scatter(i % NBUF)
            plsc.subcore_barrier()
            pltpu.sync_copy(acc_sh.at[pl.ds(s * RPT, RPT)],
                            out_hbm.at[pl.ds(chunk * N + s * RPT, RPT)])

    return _agg


def _prep_body(x_ref, degp_ref, emb_ref, g1_ref, dis_ref):
    xb = x_ref[...]
    ones32 = jnp.ones((32, 1), f32)
    deg = lax.dot_general(degp_ref[0], ones32,
                          (((0,), (0,)), ((), ())),
                          preferred_element_type=f32) + 1.0
    dis = lax.rsqrt(deg)
    ids = xb[:, 0:1].astype(i32)
    oh = (ids == jnp.arange(NUM_TYPES, dtype=i32)[None, :]).astype(f32)
    te = jnp.dot(oh, emb_ref[...], preferred_element_type=f32)
    half0 = jnp.concatenate([te, xb[:, 1:49]], axis=1)
    half1 = jnp.concatenate([xb[:, 49:128], jnp.zeros((R, 1), f32)], axis=1)
    g1_ref[0] = half0 * dis
    g1_ref[1] = half1 * dis
    dis_ref[...] = dis


_prep_call = pl.pallas_call(
    _prep_body,
    grid=(NB,),
    in_specs=[
        pl.BlockSpec((R, 128), lambda i: (i, 0)),
        pl.BlockSpec((1, 32, R), lambda i: (i, 0, 0)),
        pl.BlockSpec((NUM_TYPES, 32), lambda i: (0, 0)),
    ],
    out_specs=[
        pl.BlockSpec((2, R, 80), lambda i: (0, i, 0)),
        pl.BlockSpec((R, 1), lambda i: (i, 0)),
    ],
    out_shape=[
        jax.ShapeDtypeStruct((2, N, 80), f32),
        jax.ShapeDtypeStruct((N, 1), f32),
    ],
)


def _mm_body(a1_ref, dis_ref, W1_ref, b1_ref, W2_ref, out_ref):
    d = dis_ref[...]
    x0 = (a1_ref[0] * d).astype(jnp.bfloat16)
    x1 = (a1_ref[1] * d).astype(jnp.bfloat16)
    h1 = jnp.dot(x0, W1_ref[0:80, :], preferred_element_type=f32)
    h1 = h1 + jnp.dot(x1, W1_ref[80:160, :], preferred_element_type=f32)
    h1 = jnp.maximum(h1 + b1_ref[...], 0.0).astype(jnp.bfloat16)
    t = (jnp.dot(h1, W2_ref[...], preferred_element_type=f32) * d).astype(
        jnp.bfloat16)
    for j in range(8):
        out_ref[j] = t[:, j * 128:(j + 1) * 128]


_mm_call = pl.pallas_call(
    _mm_body,
    grid=(NB,),
    in_specs=[
        pl.BlockSpec((2, R, 80), lambda i: (0, i, 0)),
        pl.BlockSpec((R, 1), lambda i: (i, 0)),
        pl.BlockSpec((160, 2048), lambda i: (0, 0)),
        pl.BlockSpec((1, 2048), lambda i: (0, 0)),
        pl.BlockSpec((2048, 1024), lambda i: (0, 0)),
    ],
    out_specs=pl.BlockSpec((8, R, 128), lambda i: (0, i, 0)),
    out_shape=jax.ShapeDtypeStruct((8, N, 128), jnp.bfloat16),
)


def _head_body(a2_ref, dis_ref, b2_ref, batch_ref,
               Wh1_ref, bh1_ref, Wh2_ref, bh2_ref, Wo_ref, bo_ref,
               out_ref, pool_s, cnt_s):
    i = pl.program_id(0)

    @pl.when(i == 0)
    def _():
        pool_s[...] = jnp.zeros_like(pool_s)
        cnt_s[...] = jnp.zeros_like(cnt_s)

    d = dis_ref[...]
    bvals = batch_ref[0]
    oh = (bvals == jnp.arange(NUM_GRAPHS, dtype=i32)[:, None]).astype(f32)
    cnt_s[...] += jnp.broadcast_to(
        jnp.sum(oh, axis=1, keepdims=True), cnt_s.shape)
    for j in range(8):
        h2 = a2_ref[j].astype(f32) * d
        h2 = jnp.maximum(h2 + b2_ref[0, j * 128:(j + 1) * 128], 0.0)
        pool_s[:, j * 128:(j + 1) * 128] += jnp.dot(
            oh, h2, preferred_element_type=f32)

    @pl.when(i == NB - 1)
    def _():
        cnt = jnp.maximum(cnt_s[:, 0:1], 1.0)
        p = pool_s[...] / cnt
        hh = jnp.maximum(
            jnp.dot(p, Wh1_ref[...], preferred_element_type=f32)
            + bh1_ref[...], 0.0)
        hh = jnp.maximum(
            jnp.dot(hh, Wh2_ref[...], preferred_element_type=f32)
            + bh2_ref[...], 0.0)
        lo = jnp.dot(hh, Wo_ref[...], preferred_element_type=f32) + bo_ref[...]
        m = jnp.max(lo, axis=1, keepdims=True)
        e = jnp.exp(lo - m)
        out_ref[...] = e / jnp.sum(e, axis=1, keepdims=True)


_head_call = pl.pallas_call(
    _head_body,
    grid=(NB,),
    in_specs=[
        pl.BlockSpec((8, R, 128), lambda i: (0, i, 0)),
        pl.BlockSpec((R, 1), lambda i: (i, 0)),
        pl.BlockSpec((1, 1024), lambda i: (0, 0)),
        pl.BlockSpec((1, 1, R), lambda i: (i, 0, 0)),
        pl.BlockSpec((1024, 1024), lambda i: (0, 0)),
        pl.BlockSpec((1, 1024), lambda i: (0, 0)),
        pl.BlockSpec((1024, 512), lambda i: (0, 0)),
        pl.BlockSpec((1, 512), lambda i: (0, 0)),
        pl.BlockSpec((512, 10), lambda i: (0, 0)),
        pl.BlockSpec((1, 10), lambda i: (0, 0)),
    ],
    out_specs=pl.BlockSpec((NUM_GRAPHS, 10), lambda i: (0, 0)),
    out_shape=jax.ShapeDtypeStruct((NUM_GRAPHS, 10), f32),
    scratch_shapes=[
        pltpu.VMEM((NUM_GRAPHS, 1024), f32),
        pltpu.VMEM((NUM_GRAPHS, 128), f32),
    ],
)


def kernel(x, edge_index, batch, emb_table, W1, b1, W2, b2,
           Wh1, bh1, Wh2, bh2, Wo, bo):
    src = edge_index[0]
    dst = edge_index[1]

    degp = _make_deg()(dst)
    g1, dis2 = _prep_call(x, degp, emb_table)

    off1 = (jnp.arange(2, dtype=i32) * N)[:, None]
    src1 = jnp.reshape(src[None, :] + off1, (-1,))
    a1 = _make_agg(2, 80, 80, 5)(src1, dst.reshape(16, 125, 80),
                                 g1.reshape(2 * N, 80))
    a1 = a1.reshape(2, N, 80)

    W1p = jnp.concatenate([W1, jnp.zeros((1, 2048), f32)], axis=0)
    g2 = _mm_call(a1, dis2, W1p.astype(jnp.bfloat16),
                  b1.reshape(1, 2048), W2.astype(jnp.bfloat16))

    off2 = (jnp.arange(8, dtype=i32) * N)[:, None]
    src2 = jnp.reshape(src[None, :] + off2, (-1,))
    a2 = _make_agg(8, 128, 80, 5, jnp.bfloat16)(
        src2, dst.reshape(16, 125, 80), g2.reshape(8 * N, 128))
    a2 = a2.reshape(8, N, 128)

    return _head_call(a2, dis2, b2.reshape(1, 1024),
                      batch.reshape(NB, 1, R), Wh1, bh1.reshape(1, 1024),
                      Wh2, bh2.reshape(1, 512), Wo, bo.reshape(1, 10))


# speedup vs baseline: 1.1044x; 1.0067x over previous
import functools

import jax
import jax.numpy as jnp
from jax import lax
from jax.experimental import pallas as pl
from jax.experimental.pallas import tpu as pltpu
from jax.experimental.pallas import tpu_sc as plsc

N = 10000
E = 160000
NUM_TYPES = 100
NUM_GRAPHS = 16
R = 1000
NB = N // R

f32 = jnp.float32
i32 = jnp.int32


def _sc_mesh():
    return plsc.VectorSubcoreMesh(core_axis_name="c", subcore_axis_name="s")


EVEC = E // 16
VMAX = 313
EPW_PAD = VMAX * 16


@functools.cache
def _make_deg():
    @functools.partial(
        pl.kernel,
        out_type=jax.ShapeDtypeStruct((NB, 32, R), f32),
        mesh=_sc_mesh(),
        compiler_params=pltpu.CompilerParams(needs_layout_passes=False, use_tc_tiling_on_sc=False),
        scratch_types=[
            pltpu.VMEM((N,), f32),
            pltpu.VMEM((EPW_PAD,), i32),
        ],
    )
    def _deg_kernel(dst_hbm, out_hbm, acc_v, dst_v):
        c = lax.axis_index("c")
        s = lax.axis_index("s")
        wid = c * 16 + s

        def zbody(i, _):
            acc_v[pl.ds(i * 16, 16)] = jnp.zeros((16,), f32)
            return 0

        lax.fori_loop(0, N // 16, zbody, 0)

        b = 312 * wid + wid // 2
        n = 313 - (wid % 2)
        pltpu.sync_copy(dst_hbm.at[pl.ds(b * 16, EPW_PAD)], dst_v)

        ones = jnp.ones((16,), f32)

        def ebody(i, _):
            idx = dst_v[pl.ds(i * 16, 16)]
            plsc.addupdate_scatter(acc_v, [idx], ones)
            return 0

        lax.fori_loop(0, n, ebody, 0)
        for b in range(NB):
            pltpu.sync_copy(acc_v.at[pl.ds(b * R, R)], out_hbm.at[b, wid])

    return _deg_kernel


@functools.cache
def _make_agg(n_chunks, D, CH, NBUF, dtype=f32):
    per_sc = n_chunks // 2
    EPT = E // 16
    ITER = EPT // CH
    RPT = N // 16
    P = 3
    assert ITER % NBUF == 0 and CH % 8 == 0 and CH <= 128 and P < NBUF

    @functools.partial(
        pl.kernel,
        out_type=jax.ShapeDtypeStruct((n_chunks * N, D), dtype),
        mesh=_sc_mesh(),
        compiler_params=pltpu.CompilerParams(needs_layout_passes=False, use_tc_tiling_on_sc=False),
        scratch_types=[
            pltpu.VMEM((EPT,), i32),
            pltpu.VMEM((ITER, CH), i32),
            pltpu.VMEM((NBUF, CH, D), dtype),
            pltpu.VMEM_SHARED((N, D), dtype),
            pltpu.SemaphoreType.DMA((NBUF,)),
            pltpu.SemaphoreType.DMA((NBUF,)),
        ],
    )
    def _agg(src_hbm, dst_hbm, g_hbm, out_hbm,
             src_v, dst_v, rows_v, acc_sh, sem_g, sem_s):
        c = lax.axis_index("c")
        s = lax.axis_index("s")

        def gather(i, b):
            pltpu.async_copy(g_hbm.at[src_v.at[pl.ds(i * CH, CH)]],
                             rows_v.at[b], sem_g.at[b])

        def wait_gather(b):
            pltpu.make_async_copy(g_hbm.at[pl.ds(0, CH)], rows_v.at[b],
                                  sem_g.at[b]).wait()

        def scatter(i, b):
            pltpu.async_copy(rows_v.at[b], acc_sh.at[dst_v.at[i]],
                             sem_s.at[b], add=True)

        def wait_scatter(b):
            pltpu.make_async_copy(rows_v.at[b], acc_sh.at[pl.ds(0, CH)],
                                  sem_s.at[b]).wait()

        for k in range(per_sc):
            chunk = c * per_sc + k
            pltpu.sync_copy(src_hbm.at[pl.ds(chunk * E + s * EPT, EPT)], src_v)
            pltpu.sync_copy(dst_hbm.at[s], dst_v)
            for b in range(P):
                gather(b, b)
            pltpu.sync_copy(g_hbm.at[pl.ds(chunk * N + s * RPT, RPT)],
                            acc_sh.at[pl.ds(s * RPT, RPT)])
            plsc.subcore_barrier()

            def obody(io, _):
                for b in range(NBUF):
                    i = io * NBUF + b
                    bg = (b + P) % NBUF
                    wait_gather(b)
                    scatter(i, b)

                    @pl.when(i >= NBUF - P)
                    def _():
                        wait_scatter(bg)

                    @pl.when(i + P < ITER)
                    def _():
                        gather(i + P, bg)
                return 0

            lax.fori_loop(0, ITER // NBUF, obody, 0)
            for i in range(ITER - (NBUF - P), ITER):
                wait_scatter(i % NBUF)
            plsc.subcore_barrier()
            pltpu.sync_copy(acc_sh.at[pl.ds(s * RPT, RPT)],
                            out_hbm.at[pl.ds(chunk * N + s * RPT, RPT)])

    return _agg


def _prep_body(x_ref, degp_ref, emb_ref, g1_ref, dis_ref):
    xb = x_ref[...]
    ones32 = jnp.ones((32, 1), f32)
    deg = lax.dot_general(degp_ref[0], ones32,
                          (((0,), (0,)), ((), ())),
                          preferred_element_type=f32) + 1.0
    dis = lax.rsqrt(deg)
    ids = xb[:, 0:1].astype(i32)
    oh = (ids == jnp.arange(NUM_TYPES, dtype=i32)[None, :]).astype(f32)
    te = jnp.dot(oh, emb_ref[...], preferred_element_type=f32)
    half0 = jnp.concatenate([te, xb[:, 1:49]], axis=1)
    half1 = jnp.concatenate([xb[:, 49:128], jnp.zeros((R, 1), f32)], axis=1)
    g1_ref[0] = (half0 * dis).astype(jnp.bfloat16)
    g1_ref[1] = (half1 * dis).astype(jnp.bfloat16)
    dis_ref[...] = dis


_prep_call = pl.pallas_call(
    _prep_body,
    grid=(NB,),
    in_specs=[
        pl.BlockSpec((R, 128), lambda i: (i, 0)),
        pl.BlockSpec((1, 32, R), lambda i: (i, 0, 0)),
        pl.BlockSpec((NUM_TYPES, 32), lambda i: (0, 0)),
    ],
    out_specs=[
        pl.BlockSpec((2, R, 80), lambda i: (0, i, 0)),
        pl.BlockSpec((R, 1), lambda i: (i, 0)),
    ],
    out_shape=[
        jax.ShapeDtypeStruct((2, N, 80), jnp.bfloat16),
        jax.ShapeDtypeStruct((N, 1), f32),
    ],
)


def _mm_body(a1_ref, dis_ref, W1_ref, b1_ref, W2_ref, out_ref):
    d = dis_ref[...]
    x0 = (a1_ref[0].astype(f32) * d).astype(jnp.bfloat16)
    x1 = (a1_ref[1].astype(f32) * d).astype(jnp.bfloat16)
    h1 = jnp.dot(x0, W1_ref[0:80, :], preferred_element_type=f32)
    h1 = h1 + jnp.dot(x1, W1_ref[80:160, :], preferred_element_type=f32)
    h1 = jnp.maximum(h1 + b1_ref[...], 0.0).astype(jnp.bfloat16)
    t = (jnp.dot(h1, W2_ref[...], preferred_element_type=f32) * d).astype(
        jnp.bfloat16)
    for j in range(8):
        out_ref[j] = t[:, j * 128:(j + 1) * 128]


_mm_call = pl.pallas_call(
    _mm_body,
    grid=(NB,),
    in_specs=[
        pl.BlockSpec((2, R, 80), lambda i: (0, i, 0)),
        pl.BlockSpec((R, 1), lambda i: (i, 0)),
        pl.BlockSpec((160, 2048), lambda i: (0, 0)),
        pl.BlockSpec((1, 2048), lambda i: (0, 0)),
        pl.BlockSpec((2048, 1024), lambda i: (0, 0)),
    ],
    out_specs=pl.BlockSpec((8, R, 128), lambda i: (0, i, 0)),
    out_shape=jax.ShapeDtypeStruct((8, N, 128), jnp.bfloat16),
)


def _head_body(a2_ref, dis_ref, b2_ref, batch_ref,
               Wh1_ref, bh1_ref, Wh2_ref, bh2_ref, Wo_ref, bo_ref,
               out_ref, pool_s, cnt_s):
    i = pl.program_id(0)

    @pl.when(i == 0)
    def _():
        pool_s[...] = jnp.zeros_like(pool_s)
        cnt_s[...] = jnp.zeros_like(cnt_s)

    d = dis_ref[...]
    bvals = batch_ref[0]
    oh = (bvals == jnp.arange(NUM_GRAPHS, dtype=i32)[:, None]).astype(f32)
    cnt_s[...] += jnp.broadcast_to(
        jnp.sum(oh, axis=1, keepdims=True), cnt_s.shape)
    for j in range(8):
        h2 = a2_ref[j].astype(f32) * d
        h2 = jnp.maximum(h2 + b2_ref[0, j * 128:(j + 1) * 128], 0.0)
        pool_s[:, j * 128:(j + 1) * 128] += jnp.dot(
            oh, h2, preferred_element_type=f32)

    @pl.when(i == NB - 1)
    def _():
        cnt = jnp.maximum(cnt_s[:, 0:1], 1.0)
        p = pool_s[...] / cnt
        hh = jnp.maximum(
            jnp.dot(p, Wh1_ref[...], preferred_element_type=f32)
            + bh1_ref[...], 0.0)
        hh = jnp.maximum(
            jnp.dot(hh, Wh2_ref[...], preferred_element_type=f32)
            + bh2_ref[...], 0.0)
        lo = jnp.dot(hh, Wo_ref[...], preferred_element_type=f32) + bo_ref[...]
        m = jnp.max(lo, axis=1, keepdims=True)
        e = jnp.exp(lo - m)
        out_ref[...] = e / jnp.sum(e, axis=1, keepdims=True)


_head_call = pl.pallas_call(
    _head_body,
    grid=(NB,),
    in_specs=[
        pl.BlockSpec((8, R, 128), lambda i: (0, i, 0)),
        pl.BlockSpec((R, 1), lambda i: (i, 0)),
        pl.BlockSpec((1, 1024), lambda i: (0, 0)),
        pl.BlockSpec((1, 1, R), lambda i: (i, 0, 0)),
        pl.BlockSpec((1024, 1024), lambda i: (0, 0)),
        pl.BlockSpec((1, 1024), lambda i: (0, 0)),
        pl.BlockSpec((1024, 512), lambda i: (0, 0)),
        pl.BlockSpec((1, 512), lambda i: (0, 0)),
        pl.BlockSpec((512, 10), lambda i: (0, 0)),
        pl.BlockSpec((1, 10), lambda i: (0, 0)),
    ],
    out_specs=pl.BlockSpec((NUM_GRAPHS, 10), lambda i: (0, 0)),
    out_shape=jax.ShapeDtypeStruct((NUM_GRAPHS, 10), f32),
    scratch_shapes=[
        pltpu.VMEM((NUM_GRAPHS, 1024), f32),
        pltpu.VMEM((NUM_GRAPHS, 128), f32),
    ],
)


def kernel(x, edge_index, batch, emb_table, W1, b1, W2, b2,
           Wh1, bh1, Wh2, bh2, Wo, bo):
    src = edge_index[0]
    dst = edge_index[1]

    degp = _make_deg()(dst)
    g1, dis2 = _prep_call(x, degp, emb_table)

    off1 = (jnp.arange(2, dtype=i32) * N)[:, None]
    src1 = jnp.reshape(src[None, :] + off1, (-1,))
    a1 = _make_agg(2, 80, 80, 5, jnp.bfloat16)(
        src1, dst.reshape(16, 125, 80), g1.reshape(2 * N, 80))
    a1 = a1.reshape(2, N, 80)

    W1p = jnp.concatenate([W1, jnp.zeros((1, 2048), f32)], axis=0)
    g2 = _mm_call(a1, dis2, W1p.astype(jnp.bfloat16),
                  b1.reshape(1, 2048), W2.astype(jnp.bfloat16))

    off2 = (jnp.arange(8, dtype=i32) * N)[:, None]
    src2 = jnp.reshape(src[None, :] + off2, (-1,))
    a2 = _make_agg(8, 128, 80, 5, jnp.bfloat16)(
        src2, dst.reshape(16, 125, 80), g2.reshape(8 * N, 128))
    a2 = a2.reshape(8, N, 128)

    return _head_call(a2, dis2, b2.reshape(1, 1024),
                      batch.reshape(NB, 1, R), Wh1, bh1.reshape(1, 1024),
                      Wh2, bh2.reshape(1, 512), Wo, bo.reshape(1, 10))


# speedup vs baseline: 1.1530x; 1.0440x over previous
import functools

import jax
import jax.numpy as jnp
from jax import lax
from jax.experimental import pallas as pl
from jax.experimental.pallas import tpu as pltpu
from jax.experimental.pallas import tpu_sc as plsc

N = 10000
E = 160000
NUM_TYPES = 100
NUM_GRAPHS = 16
R = 1000
NB = N // R

f32 = jnp.float32
i32 = jnp.int32


def _sc_mesh():
    return plsc.VectorSubcoreMesh(core_axis_name="c", subcore_axis_name="s")


EVEC = E // 16
VMAX = 313
EPW_PAD = VMAX * 16


@functools.cache
def _make_deg():
    @functools.partial(
        pl.kernel,
        out_type=jax.ShapeDtypeStruct((NB, 32, R), f32),
        mesh=_sc_mesh(),
        compiler_params=pltpu.CompilerParams(needs_layout_passes=False, use_tc_tiling_on_sc=False),
        scratch_types=[
            pltpu.VMEM((N,), f32),
            pltpu.VMEM((EPW_PAD,), i32),
        ],
    )
    def _deg_kernel(dst_hbm, out_hbm, acc_v, dst_v):
        c = lax.axis_index("c")
        s = lax.axis_index("s")
        wid = c * 16 + s

        def zbody(i, _):
            acc_v[pl.ds(i * 16, 16)] = jnp.zeros((16,), f32)
            return 0

        lax.fori_loop(0, N // 16, zbody, 0)

        b = 312 * wid + wid // 2
        n = 313 - (wid % 2)
        pltpu.sync_copy(dst_hbm.at[pl.ds(b * 16, EPW_PAD)], dst_v)

        ones = jnp.ones((16,), f32)

        def ebody(i, _):
            idx = dst_v[pl.ds(i * 16, 16)]
            plsc.addupdate_scatter(acc_v, [idx], ones)
            return 0

        lax.fori_loop(0, n, ebody, 0)
        for b in range(NB):
            pltpu.sync_copy(acc_v.at[pl.ds(b * R, R)], out_hbm.at[b, wid])

    return _deg_kernel


@functools.cache
def _make_agg(n_chunks, D, CH, NBUF, dtype=f32):
    per_sc = n_chunks // 2
    EPT = E // 16
    ITER = EPT // CH
    RPT = N // 16
    P = 4
    assert ITER % NBUF == 0 and CH % 8 == 0 and CH <= 128 and P < NBUF

    @functools.partial(
        pl.kernel,
        out_type=jax.ShapeDtypeStruct((n_chunks * N, D), dtype),
        mesh=_sc_mesh(),
        compiler_params=pltpu.CompilerParams(needs_layout_passes=False, use_tc_tiling_on_sc=False),
        scratch_types=[
            pltpu.VMEM((EPT,), i32),
            pltpu.VMEM((ITER, CH), i32),
            pltpu.VMEM((NBUF, CH, D), dtype),
            pltpu.VMEM_SHARED((N, D), dtype),
            pltpu.SemaphoreType.DMA((NBUF,)),
            pltpu.SemaphoreType.DMA((NBUF,)),
        ],
    )
    def _agg(src_hbm, dst_hbm, g_hbm, out_hbm,
             src_v, dst_v, rows_v, acc_sh, sem_g, sem_s):
        c = lax.axis_index("c")
        s = lax.axis_index("s")

        def gather(i, b):
            pltpu.async_copy(g_hbm.at[src_v.at[pl.ds(i * CH, CH)]],
                             rows_v.at[b], sem_g.at[b])

        def wait_gather(b):
            pltpu.make_async_copy(g_hbm.at[pl.ds(0, CH)], rows_v.at[b],
                                  sem_g.at[b]).wait()

        def scatter(i, b):
            pltpu.async_copy(rows_v.at[b], acc_sh.at[dst_v.at[i]],
                             sem_s.at[b], add=True)

        def wait_scatter(b):
            pltpu.make_async_copy(rows_v.at[b], acc_sh.at[pl.ds(0, CH)],
                                  sem_s.at[b]).wait()

        for k in range(per_sc):
            chunk = c * per_sc + k
            pltpu.sync_copy(src_hbm.at[pl.ds(chunk * E + s * EPT, EPT)], src_v)
            pltpu.sync_copy(dst_hbm.at[s], dst_v)
            for b in range(P):
                gather(b, b)
            pltpu.sync_copy(g_hbm.at[pl.ds(chunk * N + s * RPT, RPT)],
                            acc_sh.at[pl.ds(s * RPT, RPT)])
            plsc.subcore_barrier()

            def obody(io, _):
                for b in range(NBUF):
                    i = io * NBUF + b
                    bg = (b + P) % NBUF
                    wait_gather(b)
                    scatter(i, b)

                    @pl.when(i >= NBUF - P)
                    def _():
                        wait_scatter(bg)

                    @pl.when(i + P < ITER)
                    def _():
                        gather(i + P, bg)
                return 0

            lax.fori_loop(0, ITER // NBUF, obody, 0)
            for i in range(ITER - (NBUF - P), ITER):
                wait_scatter(i % NBUF)
            plsc.subcore_barrier()
            pltpu.sync_copy(acc_sh.at[pl.ds(s * RPT, RPT)],
                            out_hbm.at[pl.ds(chunk * N + s * RPT, RPT)])

    return _agg


def _prep_body(x_ref, degp_ref, emb_ref, g1_ref, dis_ref):
    xb = x_ref[...]
    ones32 = jnp.ones((32, 1), f32)
    deg = lax.dot_general(degp_ref[0], ones32,
                          (((0,), (0,)), ((), ())),
                          preferred_element_type=f32) + 1.0
    dis = lax.rsqrt(deg)
    ids = xb[:, 0:1].astype(i32)
    oh = (ids == jnp.arange(NUM_TYPES, dtype=i32)[None, :]).astype(f32)
    te = jnp.dot(oh, emb_ref[...], preferred_element_type=f32)
    half0 = jnp.concatenate([te, xb[:, 1:49]], axis=1)
    half1 = jnp.concatenate([xb[:, 49:128], jnp.zeros((R, 1), f32)], axis=1)
    g1_ref[0] = (half0 * dis).astype(jnp.bfloat16)
    g1_ref[1] = (half1 * dis).astype(jnp.bfloat16)
    dis_ref[...] = dis


_prep_call = pl.pallas_call(
    _prep_body,
    grid=(NB,),
    in_specs=[
        pl.BlockSpec((R, 128), lambda i: (i, 0)),
        pl.BlockSpec((1, 32, R), lambda i: (i, 0, 0)),
        pl.BlockSpec((NUM_TYPES, 32), lambda i: (0, 0)),
    ],
    out_specs=[
        pl.BlockSpec((2, R, 80), lambda i: (0, i, 0)),
        pl.BlockSpec((R, 1), lambda i: (i, 0)),
    ],
    out_shape=[
        jax.ShapeDtypeStruct((2, N, 80), jnp.bfloat16),
        jax.ShapeDtypeStruct((N, 1), f32),
    ],
)


def _mm_body(a1_ref, dis_ref, W1_ref, b1_ref, W2_ref, out_ref):
    d = dis_ref[...]
    x0 = (a1_ref[0].astype(f32) * d).astype(jnp.bfloat16)
    x1 = (a1_ref[1].astype(f32) * d).astype(jnp.bfloat16)
    h1 = jnp.dot(x0, W1_ref[0:80, :], preferred_element_type=f32)
    h1 = h1 + jnp.dot(x1, W1_ref[80:160, :], preferred_element_type=f32)
    h1 = jnp.maximum(h1 + b1_ref[...], 0.0).astype(jnp.bfloat16)
    t = (jnp.dot(h1, W2_ref[...], preferred_element_type=f32) * d).astype(
        jnp.bfloat16)
    for j in range(8):
        out_ref[j] = t[:, j * 128:(j + 1) * 128]


_mm_call = pl.pallas_call(
    _mm_body,
    grid=(NB,),
    in_specs=[
        pl.BlockSpec((2, R, 80), lambda i: (0, i, 0)),
        pl.BlockSpec((R, 1), lambda i: (i, 0)),
        pl.BlockSpec((160, 2048), lambda i: (0, 0)),
        pl.BlockSpec((1, 2048), lambda i: (0, 0)),
        pl.BlockSpec((2048, 1024), lambda i: (0, 0)),
    ],
    out_specs=pl.BlockSpec((8, R, 128), lambda i: (0, i, 0)),
    out_shape=jax.ShapeDtypeStruct((8, N, 128), jnp.bfloat16),
)


def _head_body(a2_ref, dis_ref, b2_ref, batch_ref,
               Wh1_ref, bh1_ref, Wh2_ref, bh2_ref, Wo_ref, bo_ref,
               out_ref, pool_s, cnt_s):
    i = pl.program_id(0)

    @pl.when(i == 0)
    def _():
        pool_s[...] = jnp.zeros_like(pool_s)
        cnt_s[...] = jnp.zeros_like(cnt_s)

    d = dis_ref[...]
    bvals = batch_ref[0]
    oh = (bvals == jnp.arange(NUM_GRAPHS, dtype=i32)[:, None]).astype(f32)
    cnt_s[...] += jnp.broadcast_to(
        jnp.sum(oh, axis=1, keepdims=True), cnt_s.shape)
    for j in range(8):
        h2 = a2_ref[j].astype(f32) * d
        h2 = jnp.maximum(h2 + b2_ref[0, j * 128:(j + 1) * 128], 0.0)
        pool_s[:, j * 128:(j + 1) * 128] += jnp.dot(
            oh, h2, preferred_element_type=f32)

    @pl.when(i == NB - 1)
    def _():
        cnt = jnp.maximum(cnt_s[:, 0:1], 1.0)
        p = pool_s[...] / cnt
        hh = jnp.maximum(
            jnp.dot(p, Wh1_ref[...], preferred_element_type=f32)
            + bh1_ref[...], 0.0)
        hh = jnp.maximum(
            jnp.dot(hh, Wh2_ref[...], preferred_element_type=f32)
            + bh2_ref[...], 0.0)
        lo = jnp.dot(hh, Wo_ref[...], preferred_element_type=f32) + bo_ref[...]
        m = jnp.max(lo, axis=1, keepdims=True)
        e = jnp.exp(lo - m)
        out_ref[...] = e / jnp.sum(e, axis=1, keepdims=True)


_head_call = pl.pallas_call(
    _head_body,
    grid=(NB,),
    in_specs=[
        pl.BlockSpec((8, R, 128), lambda i: (0, i, 0)),
        pl.BlockSpec((R, 1), lambda i: (i, 0)),
        pl.BlockSpec((1, 1024), lambda i: (0, 0)),
        pl.BlockSpec((1, 1, R), lambda i: (i, 0, 0)),
        pl.BlockSpec((1024, 1024), lambda i: (0, 0)),
        pl.BlockSpec((1, 1024), lambda i: (0, 0)),
        pl.BlockSpec((1024, 512), lambda i: (0, 0)),
        pl.BlockSpec((1, 512), lambda i: (0, 0)),
        pl.BlockSpec((512, 10), lambda i: (0, 0)),
        pl.BlockSpec((1, 10), lambda i: (0, 0)),
    ],
    out_specs=pl.BlockSpec((NUM_GRAPHS, 10), lambda i: (0, 0)),
    out_shape=jax.ShapeDtypeStruct((NUM_GRAPHS, 10), f32),
    scratch_shapes=[
        pltpu.VMEM((NUM_GRAPHS, 1024), f32),
        pltpu.VMEM((NUM_GRAPHS, 128), f32),
    ],
)


def kernel(x, edge_index, batch, emb_table, W1, b1, W2, b2,
           Wh1, bh1, Wh2, bh2, Wo, bo):
    src = edge_index[0]
    dst = edge_index[1]

    degp = _make_deg()(dst)
    g1, dis2 = _prep_call(x, degp, emb_table)

    off1 = (jnp.arange(2, dtype=i32) * N)[:, None]
    src1 = jnp.reshape(src[None, :] + off1, (-1,))
    a1 = _make_agg(2, 80, 80, 5, jnp.bfloat16)(
        src1, dst.reshape(16, 125, 80), g1.reshape(2 * N, 80))
    a1 = a1.reshape(2, N, 80)

    W1p = jnp.concatenate([W1, jnp.zeros((1, 2048), f32)], axis=0)
    g2 = _mm_call(a1, dis2, W1p.astype(jnp.bfloat16),
                  b1.reshape(1, 2048), W2.astype(jnp.bfloat16))

    off2 = (jnp.arange(8, dtype=i32) * N)[:, None]
    src2 = jnp.reshape(src[None, :] + off2, (-1,))
    a2 = _make_agg(8, 128, 80, 5, jnp.bfloat16)(
        src2, dst.reshape(16, 125, 80), g2.reshape(8 * N, 128))
    a2 = a2.reshape(8, N, 128)

    return _head_call(a2, dis2, b2.reshape(1, 1024),
                      batch.reshape(NB, 1, R), Wh1, bh1.reshape(1, 1024),
                      Wh2, bh2.reshape(1, 512), Wo, bo.reshape(1, 10))
